# Initial kernel scaffold; baseline (speedup 1.0000x reference)
import jax
import jax.numpy as jnp
from jax.experimental import pallas as pl


def kernel(logits):
    raise NotImplementedError("write your pallas kernel here")


# speedup vs baseline: 4.4274x; 4.4274x over previous
import functools

import jax
import jax.numpy as jnp
from jax import lax
from jax.experimental import pallas as pl

_K = 64
_S = 4


def _select_body(x_ref, o_ref, *, k, s, value_iters, index_iters):
    x = x_ref[...]
    r_rows, n = x.shape
    b = lax.bitcast_convert_type(x, jnp.int32)
    key = jnp.where(b >= 0, b, b ^ jnp.int32(0x7FFFFFFF))

    i32 = jnp.int32
    lo0 = jnp.full((r_rows, 1), jnp.iinfo(jnp.int32).min, i32)
    hi0 = jnp.full((r_rows, 1), jnp.iinfo(jnp.int32).max, i32)

    def vstep(_, carry):
        lo, hi = carry
        mid = (lo >> 1) + (hi >> 1) + (lo & hi & 1)
        cnt = jnp.sum((key > mid).astype(i32), axis=1, keepdims=True)
        ge = cnt >= k
        return jnp.where(ge, mid + 1, lo), jnp.where(ge, hi, mid)

    lo, _ = lax.fori_loop(0, value_iters, vstep, (lo0, hi0))
    v = lo
    gt = key > v
    eq = key == v
    cgt = jnp.sum(gt.astype(i32), axis=1, keepdims=True)
    need = k - cgt

    idx = lax.broadcasted_iota(i32, (r_rows, n), 1)
    lo2 = jnp.zeros((r_rows, 1), i32)
    hi2 = jnp.full((r_rows, 1), n - 1, i32)

    def istep(_, carry):
        lo2, hi2 = carry
        mid = (lo2 + hi2) >> 1
        cnt = jnp.sum((eq & (idx <= mid)).astype(i32), axis=1, keepdims=True)
        ge = cnt >= need
        return jnp.where(ge, lo2, mid + 1), jnp.where(ge, mid, hi2)

    lo2, _ = lax.fori_loop(0, index_iters, istep, (lo2, hi2))
    mask = gt | (eq & (idx <= lo2))
    khot = jnp.where(mask, jnp.float32(1.0), jnp.float32(0.0))
    o_ref[...] = jnp.broadcast_to(khot[None], (s, r_rows, n))


def _khot(x, k, s, rows_per_block):
    bsz, n = x.shape
    grid = bsz // rows_per_block
    value_iters = 32
    index_iters = max(1, (n - 1).bit_length())
    body = functools.partial(
        _select_body, k=k, s=s, value_iters=value_iters, index_iters=index_iters
    )
    return pl.pallas_call(
        body,
        grid=(grid,),
        in_specs=[pl.BlockSpec((rows_per_block, n), lambda i: (i, 0))],
        out_specs=pl.BlockSpec((s, rows_per_block, n), lambda i: (0, i, 0)),
        out_shape=jax.ShapeDtypeStruct((s, bsz, n), jnp.float32),
    )(x)


def kernel(logits):
    bsz, n, _ = logits.shape
    x = jnp.squeeze(logits, axis=-1)
    rows_per_block = 16 if bsz % 16 == 0 else bsz
    out = _khot(x, _K, _S, rows_per_block)
    return out.reshape(_S, bsz, n, 1)


# speedup vs baseline: 6.0149x; 1.3586x over previous
import functools

import jax
import jax.numpy as jnp
from jax import lax
from jax.experimental import pallas as pl

_K = 64
_S = 4


def _select_body(x_ref, o_ref, *, k, s, value_iters, index_iters):
    x = x_ref[...]
    r_rows, n = x.shape
    b = lax.bitcast_convert_type(x, jnp.int32)
    key = jnp.where(b >= 0, b, b ^ jnp.int32(0x7FFFFFFF))

    i32 = jnp.int32
    kc = key.reshape(r_rows, k, n // k)
    cmax = jnp.max(kc, axis=2)
    lb = jnp.min(cmax, axis=1, keepdims=True)
    ub = jnp.max(cmax, axis=1, keepdims=True)

    def vcond(carry):
        lo, hi = carry
        return jnp.any(lo < hi)

    def vstep(carry):
        lo, hi = carry
        mid = (lo >> 1) + (hi >> 1) + (lo & hi & 1)
        cnt = jnp.sum((key > mid).astype(i32), axis=1, keepdims=True)
        ge = cnt >= k
        return jnp.where(ge, mid + 1, lo), jnp.where(ge, hi, mid)

    lo, _ = lax.while_loop(vcond, vstep, (lb, ub))
    v = lo
    gt = key > v
    eq = key == v
    cgt = jnp.sum(gt.astype(i32), axis=1, keepdims=True)
    ceq = jnp.sum(eq.astype(i32), axis=1, keepdims=True)
    need = k - cgt

    idx = lax.broadcasted_iota(i32, (r_rows, n), 1)
    tie = ceq > need
    lo2 = jnp.where(tie, 0, n - 1)
    hi2 = jnp.full((r_rows, 1), n - 1, i32)

    def icond(carry):
        lo2, hi2 = carry
        return jnp.any(lo2 < hi2)

    def istep(carry):
        lo2, hi2 = carry
        mid = (lo2 + hi2) >> 1
        cnt = jnp.sum((eq & (idx <= mid)).astype(i32), axis=1, keepdims=True)
        ge = cnt >= need
        return jnp.where(ge, lo2, mid + 1), jnp.where(ge, mid, hi2)

    lo2, _ = lax.while_loop(icond, istep, (lo2, hi2))
    mask = gt | (eq & (idx <= lo2))
    khot = jnp.where(mask, jnp.float32(1.0), jnp.float32(0.0))
    o_ref[...] = jnp.broadcast_to(khot[None], (s, r_rows, n))


def _khot(x, k, s, rows_per_block):
    bsz, n = x.shape
    grid = bsz // rows_per_block
    value_iters = 32
    index_iters = max(1, (n - 1).bit_length())
    body = functools.partial(
        _select_body, k=k, s=s, value_iters=value_iters, index_iters=index_iters
    )
    return pl.pallas_call(
        body,
        grid=(grid,),
        in_specs=[pl.BlockSpec((rows_per_block, n), lambda i: (i, 0))],
        out_specs=pl.BlockSpec((s, rows_per_block, n), lambda i: (0, i, 0)),
        out_shape=jax.ShapeDtypeStruct((s, bsz, n), jnp.float32),
    )(x)


def kernel(logits):
    bsz, n, _ = logits.shape
    x = jnp.squeeze(logits, axis=-1)
    rows_per_block = 16 if bsz % 16 == 0 else bsz
    out = _khot(x, _K, _S, rows_per_block)
    return out.reshape(_S, bsz, n, 1)
